# baseline (speedup 1.0000x reference)
import dataclasses
import functools

import jax
import jax.numpy as jnp
from jax import lax
from jax.experimental import pallas as pl
from jax.experimental.pallas import tpu as pltpu
from jax.experimental.pallas import tpu_sc as plsc

ROWS = 64
COLS = 32768
TOPK = 512
LANES = 16
CHUNKS = COLS // LANES
GROUPS = CHUNKS // 4

NSEG = 4
SEG = COLS // NSEG
SEGGRP = SEG // 64
RCAP = 4096
RSTRIDE = RCAP + 4 * LANES

_T0 = 2.0
_T0_BITS = 0x40000000
_MIN_I32 = -(2**31)
_HI_KEY = 0x7F800000


def _keys_of_bits(bits):
    sgn = lax.shift_right_arithmetic(bits, 31)
    flip = lax.shift_right_logical(sgn, 1)
    return lax.bitwise_xor(bits, flip)


def _kernel_body(x_hbm, o_hbm, buf0, buf1, keys, si0, si1, so0, so1):
    cid = lax.axis_index("c")
    sid = lax.axis_index("s")
    wid = sid * 2 + cid
    r0 = wid * 2

    cp_in0 = pltpu.async_copy(x_hbm.at[r0], buf0, si0)
    cp_in1 = pltpu.async_copy(x_hbm.at[r0 + 1], buf1, si1)

    def count_region(base, ngroups, mid_vec, acc0):
        one = jnp.full((LANES,), jnp.int32(1))
        zero = jnp.zeros((LANES,), jnp.int32)

        def cit(j, acc):
            a0, a1 = acc
            b = base + j * 64
            k0 = keys[pl.ds(b, LANES)]
            k1 = keys[pl.ds(b + 16, LANES)]
            k2 = keys[pl.ds(b + 32, LANES)]
            k3 = keys[pl.ds(b + 48, LANES)]
            a0 = a0 + jnp.where(k0 >= mid_vec, one, zero)
            a1 = a1 + jnp.where(k1 >= mid_vec, one, zero)
            a0 = a0 + jnp.where(k2 >= mid_vec, one, zero)
            a1 = a1 + jnp.where(k3 >= mid_vec, one, zero)
            return (a0, a1)

        a0, a1 = lax.fori_loop(0, ngroups, cit, (acc0, jnp.zeros((LANES,), jnp.int32)))
        return a0 + a1

    def find_kth(bases_ngroups, lo0, hi0):
        def cond(c):
            lo, hi = c
            return hi - lo > 1

        def body(c):
            lo, hi = c
            mid = (
                lax.shift_right_arithmetic(lo, 1)
                + lax.shift_right_arithmetic(hi, 1)
                + (lo & hi & 1)
            )
            mid_vec = jnp.full((LANES,), mid)
            acc = jnp.zeros((LANES,), jnp.int32)
            for base, ng in bases_ngroups:
                acc = count_region(base, ng, mid_vec, acc)
            big = jnp.sum(acc) >= TOPK
            return (jnp.where(big, mid, lo), jnp.where(big, hi, mid))

        lo, _ = lax.while_loop(cond, body, (lo0, hi0))
        return lo

    def process(buf, row, sem_out):
        t0_vec = jnp.full((LANES,), jnp.float32(_T0))
        zero_vec = jnp.zeros((LANES,), jnp.int32)

        def comp_it(i, carry):
            p0, p1, p2, p3, mx = carry
            ptrs = [p0, p1, p2, p3]
            base = i * 64
            for u in range(4):
                for r in range(NSEG):
                    v = buf[pl.ds(r * SEG + base + u * LANES, LANES)]
                    m = v > t0_vec
                    kb = lax.bitcast_convert_type(v, jnp.int32)
                    plsc.store_compressed(
                        keys.at[pl.ds(ptrs[r], LANES)], kb, mask=m
                    )
                    pc = plsc.all_reduce_population_count(m)
                    ptrs[r] = ptrs[r] + pc[0]
                    mx = jnp.maximum(mx, v)
            return (*ptrs, mx)

        init = tuple(jnp.int32(r * RSTRIDE) for r in range(NSEG)) + (
            jnp.full((LANES,), jnp.float32(_T0)),
        )
        *ptrs, mxv = lax.fori_loop(0, SEGGRP, comp_it, init)
        counts = [ptrs[r] - r * RSTRIDE for r in range(NSEG)]
        c_tot = counts[0] + counts[1] + counts[2] + counts[3]
        c_max = jnp.maximum(
            jnp.maximum(counts[0], counts[1]), jnp.maximum(counts[2], counts[3])
        )

        def fast_fill():
            for r in range(NSEG):
                for u in range(4):
                    keys[pl.ds(ptrs[r] + u * LANES, LANES)] = zero_vec
            regions = [
                (r * RSTRIDE, (counts[r] + 63) >> 6) for r in range(NSEG)
            ]
            hi0 = lax.bitcast_convert_type(jnp.max(mxv), jnp.int32) + 1
            kth = find_kth(regions, jnp.int32(_T0_BITS), hi0)
            kth_vec = jnp.full((LANES,), kth)
            zf = jnp.zeros((LANES,), jnp.float32)

            @pl.loop(0, GROUPS)
            def _(i):
                base = i * 64
                for u in range(4):
                    sl = pl.ds(base + u * LANES, LANES)
                    v = buf[sl]
                    bits = lax.bitcast_convert_type(v, jnp.int32)
                    buf[sl] = jnp.where(bits >= kth_vec, v, zf)

        def fallback_fill():
            @pl.loop(0, GROUPS)
            def _(i):
                base = i * 64
                for u in range(4):
                    sl = pl.ds(base + u * LANES, LANES)
                    bits = lax.bitcast_convert_type(buf[sl], jnp.int32)
                    keys[sl] = _keys_of_bits(bits)

            kth = find_kth(
                [(0, GROUPS)], jnp.int32(_MIN_I32 + 1), jnp.int32(_HI_KEY)
            )
            kth_vec = jnp.full((LANES,), kth)
            zf = jnp.zeros((LANES,), jnp.float32)

            @pl.loop(0, GROUPS)
            def _(i):
                base = i * 64
                for u in range(4):
                    sl = pl.ds(base + u * LANES, LANES)
                    v = buf[sl]
                    bits = lax.bitcast_convert_type(v, jnp.int32)
                    buf[sl] = jnp.where(_keys_of_bits(bits) >= kth_vec, v, zf)

        lax.cond(
            jnp.logical_or(c_tot < TOPK, c_max > RCAP),
            fallback_fill,
            fast_fill,
        )
        return pltpu.async_copy(buf, o_hbm.at[row], sem_out)

    cp_in0.wait()
    cp_out0 = process(buf0, r0, so0)
    cp_in1.wait()
    cp_out1 = process(buf1, r0 + 1, so1)
    cp_out0.wait()
    cp_out1.wait()


def kernel(x):
    mesh = plsc.VectorSubcoreMesh(core_axis_name="c", subcore_axis_name="s")
    cp = pltpu.CompilerParams()
    if "needs_layout_passes" in pltpu.CompilerParams.__dataclass_fields__:
        cp = dataclasses.replace(cp, needs_layout_passes=False)
    run = pl.kernel(
        _kernel_body,
        out_type=jax.ShapeDtypeStruct((ROWS, COLS), jnp.float32),
        mesh=mesh,
        compiler_params=cp,
        scratch_types=[
            pltpu.VMEM((COLS,), jnp.float32),
            pltpu.VMEM((COLS,), jnp.float32),
            pltpu.VMEM((COLS + 4 * LANES,), jnp.int32),
            pltpu.SemaphoreType.DMA,
            pltpu.SemaphoreType.DMA,
            pltpu.SemaphoreType.DMA,
            pltpu.SemaphoreType.DMA,
        ],
    )
    return run(x)


# speedup vs baseline: 1.0147x; 1.0147x over previous
import dataclasses
import functools

import jax
import jax.numpy as jnp
from jax import lax
from jax.experimental import pallas as pl
from jax.experimental.pallas import tpu as pltpu
from jax.experimental.pallas import tpu_sc as plsc

ROWS = 64
COLS = 32768
TOPK = 512
LANES = 16
CHUNKS = COLS // LANES
GROUPS = CHUNKS // 4

NSEG = 4
SEG = COLS // NSEG
SEGGRP = SEG // 64
RCAP = 4096
RSTRIDE = RCAP + 4 * LANES

_T0 = 2.0
_T0_BITS = 0x40000000
_MIN_I32 = -(2**31)
_HI_KEY = 0x7F800000


def _keys_of_bits(bits):
    sgn = lax.shift_right_arithmetic(bits, 31)
    flip = lax.shift_right_logical(sgn, 1)
    return lax.bitwise_xor(bits, flip)


def _kernel_body(x_hbm, o_hbm, buf0, buf1, keys, si0, si1, so0, so1):
    cid = lax.axis_index("c")
    sid = lax.axis_index("s")
    wid = sid * 2 + cid
    r0 = wid * 2

    cp_in0 = pltpu.async_copy(x_hbm.at[r0], buf0, si0)
    cp_in1 = pltpu.async_copy(x_hbm.at[r0 + 1], buf1, si1)

    def count_region(base, ngroups, mid_vec, acc0):
        one = jnp.full((LANES,), jnp.int32(1))
        zero = jnp.zeros((LANES,), jnp.int32)

        def cit(j, acc):
            a0, a1 = acc
            b = base + j * 64
            k0 = keys[pl.ds(b, LANES)]
            k1 = keys[pl.ds(b + 16, LANES)]
            k2 = keys[pl.ds(b + 32, LANES)]
            k3 = keys[pl.ds(b + 48, LANES)]
            a0 = a0 + jnp.where(k0 >= mid_vec, one, zero)
            a1 = a1 + jnp.where(k1 >= mid_vec, one, zero)
            a0 = a0 + jnp.where(k2 >= mid_vec, one, zero)
            a1 = a1 + jnp.where(k3 >= mid_vec, one, zero)
            return (a0, a1)

        a0, a1 = lax.fori_loop(0, ngroups, cit, (acc0, jnp.zeros((LANES,), jnp.int32)))
        return a0 + a1

    def find_kth(bases_ngroups, lo0, hi0):
        def cond(c):
            lo, hi = c
            return hi - lo > 1

        def body(c):
            lo, hi = c
            mid = (
                lax.shift_right_arithmetic(lo, 1)
                + lax.shift_right_arithmetic(hi, 1)
                + (lo & hi & 1)
            )
            mid_vec = jnp.full((LANES,), mid)
            acc = jnp.zeros((LANES,), jnp.int32)
            for base, ng in bases_ngroups:
                acc = count_region(base, ng, mid_vec, acc)
            big = jnp.sum(acc) >= TOPK
            return (jnp.where(big, mid, lo), jnp.where(big, hi, mid))

        lo, _ = lax.while_loop(cond, body, (lo0, hi0))
        return lo

    def process(buf, row, sem_out):
        t0_vec = jnp.full((LANES,), jnp.float32(_T0))
        zero_vec = jnp.zeros((LANES,), jnp.int32)

        def comp_it(i, carry):
            ptrs = list(carry)
            base = i * 64
            for u in range(4):
                for r in range(NSEG):
                    v = buf[pl.ds(r * SEG + base + u * LANES, LANES)]
                    m = v > t0_vec
                    kb = lax.bitcast_convert_type(v, jnp.int32)
                    plsc.store_compressed(
                        keys.at[pl.ds(ptrs[r], LANES)], kb, mask=m
                    )
                    pc = plsc.all_reduce_population_count(m)
                    ptrs[r] = ptrs[r] + pc[0]
            return tuple(ptrs)

        init = tuple(jnp.int32(r * RSTRIDE) for r in range(NSEG))
        ptrs = list(lax.fori_loop(0, SEGGRP, comp_it, init))
        counts = [ptrs[r] - r * RSTRIDE for r in range(NSEG)]
        c_tot = counts[0] + counts[1] + counts[2] + counts[3]
        c_max = jnp.maximum(
            jnp.maximum(counts[0], counts[1]), jnp.maximum(counts[2], counts[3])
        )

        def fast_fill():
            for r in range(NSEG):
                for u in range(4):
                    keys[pl.ds(ptrs[r] + u * LANES, LANES)] = zero_vec
            regions = [
                (r * RSTRIDE, (counts[r] + 63) >> 6) for r in range(NSEG)
            ]
            kth = find_kth(regions, jnp.int32(_T0_BITS), jnp.int32(_HI_KEY))
            kf_vec = jnp.full((LANES,), lax.bitcast_convert_type(kth, jnp.float32))
            zf = jnp.zeros((LANES,), jnp.float32)

            @pl.loop(0, GROUPS // 2)
            def _(i):
                base = i * 128
                for u in range(8):
                    sl = pl.ds(base + u * LANES, LANES)
                    v = buf[sl]
                    buf[sl] = jnp.where(v >= kf_vec, v, zf)

        def fallback_fill():
            @pl.loop(0, GROUPS)
            def _(i):
                base = i * 64
                for u in range(4):
                    sl = pl.ds(base + u * LANES, LANES)
                    bits = lax.bitcast_convert_type(buf[sl], jnp.int32)
                    keys[sl] = _keys_of_bits(bits)

            kth = find_kth(
                [(0, GROUPS)], jnp.int32(_MIN_I32 + 1), jnp.int32(_HI_KEY)
            )
            kth_vec = jnp.full((LANES,), kth)
            zf = jnp.zeros((LANES,), jnp.float32)

            @pl.loop(0, GROUPS)
            def _(i):
                base = i * 64
                for u in range(4):
                    sl = pl.ds(base + u * LANES, LANES)
                    v = buf[sl]
                    bits = lax.bitcast_convert_type(v, jnp.int32)
                    buf[sl] = jnp.where(_keys_of_bits(bits) >= kth_vec, v, zf)

        lax.cond(
            jnp.logical_or(c_tot < TOPK, c_max > RCAP),
            fallback_fill,
            fast_fill,
        )
        return pltpu.async_copy(buf, o_hbm.at[row], sem_out)

    cp_in0.wait()
    cp_out0 = process(buf0, r0, so0)
    cp_in1.wait()
    cp_out1 = process(buf1, r0 + 1, so1)
    cp_out0.wait()
    cp_out1.wait()


def kernel(x):
    mesh = plsc.VectorSubcoreMesh(core_axis_name="c", subcore_axis_name="s")
    cp = pltpu.CompilerParams()
    if "needs_layout_passes" in pltpu.CompilerParams.__dataclass_fields__:
        cp = dataclasses.replace(cp, needs_layout_passes=False)
    run = pl.kernel(
        _kernel_body,
        out_type=jax.ShapeDtypeStruct((ROWS, COLS), jnp.float32),
        mesh=mesh,
        compiler_params=cp,
        scratch_types=[
            pltpu.VMEM((COLS,), jnp.float32),
            pltpu.VMEM((COLS,), jnp.float32),
            pltpu.VMEM((COLS + 4 * LANES,), jnp.int32),
            pltpu.SemaphoreType.DMA,
            pltpu.SemaphoreType.DMA,
            pltpu.SemaphoreType.DMA,
            pltpu.SemaphoreType.DMA,
        ],
    )
    return run(x)


# speedup vs baseline: 1.0165x; 1.0018x over previous
import dataclasses
import functools

import jax
import jax.numpy as jnp
from jax import lax
from jax.experimental import pallas as pl
from jax.experimental.pallas import tpu as pltpu
from jax.experimental.pallas import tpu_sc as plsc

ROWS = 64
COLS = 32768
TOPK = 512
LANES = 16
CHUNKS = COLS // LANES
GROUPS = CHUNKS // 4

NSEG = 4
SEG = COLS // NSEG
SEGGRP = SEG // 64
RCAP = 4096
RSTRIDE = RCAP + 4 * LANES

_T0 = 2.0
_T0_BITS = 0x40000000
_MIN_I32 = -(2**31)
_HI_KEY = 0x7F800000


def _keys_of_bits(bits):
    sgn = lax.shift_right_arithmetic(bits, 31)
    flip = lax.shift_right_logical(sgn, 1)
    return lax.bitwise_xor(bits, flip)


def _kernel_body(x_hbm, o_hbm, buf0, buf1, keys, si0, si1, so0, so1):
    cid = lax.axis_index("c")
    sid = lax.axis_index("s")
    wid = sid * 2 + cid
    r0 = wid * 2

    cp_in0 = pltpu.async_copy(x_hbm.at[r0], buf0, si0)
    cp_in1 = pltpu.async_copy(x_hbm.at[r0 + 1], buf1, si1)

    def count_region(base, ngroups, midf_vec, acc0):
        one = jnp.full((LANES,), jnp.int32(1))
        zero = jnp.zeros((LANES,), jnp.int32)

        def cit(j, acc):
            a0, a1 = acc
            b = base + j * 64
            k0 = keys[pl.ds(b, LANES)]
            k1 = keys[pl.ds(b + 16, LANES)]
            k2 = keys[pl.ds(b + 32, LANES)]
            k3 = keys[pl.ds(b + 48, LANES)]
            a0 = a0 + jnp.where(k0 >= midf_vec, one, zero)
            a1 = a1 + jnp.where(k1 >= midf_vec, one, zero)
            a0 = a0 + jnp.where(k2 >= midf_vec, one, zero)
            a1 = a1 + jnp.where(k3 >= midf_vec, one, zero)
            return (a0, a1)

        a0, a1 = lax.fori_loop(0, ngroups, cit, (acc0, jnp.zeros((LANES,), jnp.int32)))
        return a0 + a1

    def find_kth(bases_ngroups, lo0, hi0):
        def cond(c):
            lo, hi = c
            return hi - lo > 1

        def body(c):
            lo, hi = c
            mid = (
                lax.shift_right_arithmetic(lo, 1)
                + lax.shift_right_arithmetic(hi, 1)
                + (lo & hi & 1)
            )
            midf_vec = jnp.full(
                (LANES,), lax.bitcast_convert_type(mid, jnp.float32)
            )
            acc = jnp.zeros((LANES,), jnp.int32)
            for base, ng in bases_ngroups:
                acc = count_region(base, ng, midf_vec, acc)
            big = jnp.sum(acc) >= TOPK
            return (jnp.where(big, mid, lo), jnp.where(big, hi, mid))

        lo, _ = lax.while_loop(cond, body, (lo0, hi0))
        return lo

    def process(buf, row, sem_out):
        t0_vec = jnp.full((LANES,), jnp.float32(_T0))
        zero_vec = jnp.zeros((LANES,), jnp.float32)

        def comp_it(i, carry):
            ptrs = list(carry)
            base = i * 64
            for u in range(4):
                for r in range(NSEG):
                    v = buf[pl.ds(r * SEG + base + u * LANES, LANES)]
                    m = v > t0_vec
                    plsc.store_compressed(
                        keys.at[pl.ds(ptrs[r], LANES)], v, mask=m
                    )
                    pc = plsc.all_reduce_population_count(m)
                    ptrs[r] = ptrs[r] + pc[0]
            return tuple(ptrs)

        init = tuple(jnp.int32(r * RSTRIDE) for r in range(NSEG))
        ptrs = list(lax.fori_loop(0, SEGGRP, comp_it, init))
        counts = [ptrs[r] - r * RSTRIDE for r in range(NSEG)]
        c_tot = counts[0] + counts[1] + counts[2] + counts[3]
        c_max = jnp.maximum(
            jnp.maximum(counts[0], counts[1]), jnp.maximum(counts[2], counts[3])
        )

        def fast_fill():
            for r in range(NSEG):
                for u in range(4):
                    keys[pl.ds(ptrs[r] + u * LANES, LANES)] = zero_vec
            regions = [
                (r * RSTRIDE, (counts[r] + 63) >> 6) for r in range(NSEG)
            ]
            kth = find_kth(regions, jnp.int32(_T0_BITS), jnp.int32(_HI_KEY))
            kf_vec = jnp.full((LANES,), lax.bitcast_convert_type(kth, jnp.float32))
            zf = jnp.zeros((LANES,), jnp.float32)

            @pl.loop(0, GROUPS // 2)
            def _(i):
                base = i * 128
                for u in range(8):
                    sl = pl.ds(base + u * LANES, LANES)
                    v = buf[sl]
                    buf[sl] = jnp.where(v >= kf_vec, v, zf)

        def fallback_fill():
            def fcond(c):
                lo, hi = c
                return hi - lo > 1

            def fbody(c):
                lo, hi = c
                mid = (
                    lax.shift_right_arithmetic(lo, 1)
                    + lax.shift_right_arithmetic(hi, 1)
                    + (lo & hi & 1)
                )
                mid_vec = jnp.full((LANES,), mid)
                one = jnp.full((LANES,), jnp.int32(1))
                zero = jnp.zeros((LANES,), jnp.int32)

                def cit(j, acc):
                    base = j * 64
                    for u in range(4):
                        bits = lax.bitcast_convert_type(
                            buf[pl.ds(base + u * LANES, LANES)], jnp.int32
                        )
                        acc = acc + jnp.where(
                            _keys_of_bits(bits) >= mid_vec, one, zero
                        )
                    return acc

                acc = lax.fori_loop(
                    0, GROUPS, cit, jnp.zeros((LANES,), jnp.int32)
                )
                big = jnp.sum(acc) >= TOPK
                return (jnp.where(big, mid, lo), jnp.where(big, hi, mid))

            kth, _ = lax.while_loop(
                fcond, fbody, (jnp.int32(_MIN_I32 + 1), jnp.int32(_HI_KEY))
            )
            kth_vec = jnp.full((LANES,), kth)
            zf = jnp.zeros((LANES,), jnp.float32)

            @pl.loop(0, GROUPS)
            def _(i):
                base = i * 64
                for u in range(4):
                    sl = pl.ds(base + u * LANES, LANES)
                    v = buf[sl]
                    bits = lax.bitcast_convert_type(v, jnp.int32)
                    buf[sl] = jnp.where(_keys_of_bits(bits) >= kth_vec, v, zf)

        lax.cond(
            jnp.logical_or(c_tot < TOPK, c_max > RCAP),
            fallback_fill,
            fast_fill,
        )
        return pltpu.async_copy(buf, o_hbm.at[row], sem_out)

    cp_in0.wait()
    cp_out0 = process(buf0, r0, so0)
    cp_in1.wait()
    cp_out1 = process(buf1, r0 + 1, so1)
    cp_out0.wait()
    cp_out1.wait()


def kernel(x):
    mesh = plsc.VectorSubcoreMesh(core_axis_name="c", subcore_axis_name="s")
    cp = pltpu.CompilerParams()
    if "needs_layout_passes" in pltpu.CompilerParams.__dataclass_fields__:
        cp = dataclasses.replace(cp, needs_layout_passes=False)
    run = pl.kernel(
        _kernel_body,
        out_type=jax.ShapeDtypeStruct((ROWS, COLS), jnp.float32),
        mesh=mesh,
        compiler_params=cp,
        scratch_types=[
            pltpu.VMEM((COLS,), jnp.float32),
            pltpu.VMEM((COLS,), jnp.float32),
            pltpu.VMEM(((NSEG - 1) * RSTRIDE + SEG + 4 * LANES,), jnp.float32),
            pltpu.SemaphoreType.DMA,
            pltpu.SemaphoreType.DMA,
            pltpu.SemaphoreType.DMA,
            pltpu.SemaphoreType.DMA,
        ],
    )
    return run(x)


# speedup vs baseline: 1.0401x; 1.0232x over previous
import dataclasses
import functools

import jax
import jax.numpy as jnp
from jax import lax
from jax.experimental import pallas as pl
from jax.experimental.pallas import tpu as pltpu
from jax.experimental.pallas import tpu_sc as plsc

ROWS = 64
COLS = 32768
TOPK = 512
LANES = 16
CHUNKS = COLS // LANES
GROUPS = CHUNKS // 4

NSEG = 4
SEG = COLS // NSEG
SEGGRP = SEG // 64
RCAP = 4096
RSTRIDE = RCAP + 4 * LANES

_T0 = 2.0
_T0_BITS = 0x40000000
_MIN_I32 = -(2**31)
_HI_KEY = 0x7F800000


def _keys_of_bits(bits):
    sgn = lax.shift_right_arithmetic(bits, 31)
    flip = lax.shift_right_logical(sgn, 1)
    return lax.bitwise_xor(bits, flip)


def _kernel_body(x_hbm, o_hbm, buf0, buf1, keys, si0, si1, so0, so1):
    cid = lax.axis_index("c")
    sid = lax.axis_index("s")
    wid = sid * 2 + cid
    r0 = wid * 2

    cp_in0 = pltpu.async_copy(x_hbm.at[r0], buf0, si0)
    cp_in1 = pltpu.async_copy(x_hbm.at[r0 + 1], buf1, si1)

    def count_region(base, ngroups, midf_vec, acc0):
        one = jnp.full((LANES,), jnp.int32(1))
        zero = jnp.zeros((LANES,), jnp.int32)

        def cit(j, acc):
            a0, a1 = acc
            b = base + j * 64
            k0 = keys[pl.ds(b, LANES)]
            k1 = keys[pl.ds(b + 16, LANES)]
            k2 = keys[pl.ds(b + 32, LANES)]
            k3 = keys[pl.ds(b + 48, LANES)]
            a0 = a0 + jnp.where(k0 >= midf_vec, one, zero)
            a1 = a1 + jnp.where(k1 >= midf_vec, one, zero)
            a0 = a0 + jnp.where(k2 >= midf_vec, one, zero)
            a1 = a1 + jnp.where(k3 >= midf_vec, one, zero)
            return (a0, a1)

        a0, a1 = lax.fori_loop(0, ngroups, cit, (acc0, jnp.zeros((LANES,), jnp.int32)))
        return a0 + a1

    def find_kth(bases_ngroups, lo0, hi0):
        def cond(c):
            lo, hi = c
            return hi - lo > 1

        def body(c):
            lo, hi = c
            mid = (
                lax.shift_right_arithmetic(lo, 1)
                + lax.shift_right_arithmetic(hi, 1)
                + (lo & hi & 1)
            )
            midf_vec = jnp.full(
                (LANES,), lax.bitcast_convert_type(mid, jnp.float32)
            )
            acc = jnp.zeros((LANES,), jnp.int32)
            for base, ng in bases_ngroups:
                acc = count_region(base, ng, midf_vec, acc)
            big = jnp.sum(acc) >= TOPK
            return (jnp.where(big, mid, lo), jnp.where(big, hi, mid))

        lo, _ = lax.while_loop(cond, body, (lo0, hi0))
        return lo

    def process(buf, row, sem_out):
        t0_vec = jnp.full((LANES,), jnp.float32(_T0))
        zero_vec = jnp.zeros((LANES,), jnp.float32)

        def comp_it(i, carry):
            ptrs = list(carry)
            base = i * 64
            for u in range(4):
                for r in range(NSEG):
                    v = buf[pl.ds(r * SEG + base + u * LANES, LANES)]
                    m = v > t0_vec
                    plsc.store_compressed(
                        keys.at[pl.ds(ptrs[r], LANES)], v, mask=m
                    )
                    pc = plsc.all_reduce_population_count(m)
                    ptrs[r] = ptrs[r] + pc[0]
            return tuple(ptrs)

        init = tuple(jnp.int32(r * RSTRIDE) for r in range(NSEG))
        ptrs = list(lax.fori_loop(0, SEGGRP, comp_it, init))
        counts = [ptrs[r] - r * RSTRIDE for r in range(NSEG)]
        c_tot = counts[0] + counts[1] + counts[2] + counts[3]
        c_max = jnp.maximum(
            jnp.maximum(counts[0], counts[1]), jnp.maximum(counts[2], counts[3])
        )

        def fast_kth():
            for r in range(NSEG):
                for u in range(4):
                    keys[pl.ds(ptrs[r] + u * LANES, LANES)] = zero_vec
            regions = [
                (r * RSTRIDE, (counts[r] + 63) >> 6) for r in range(NSEG)
            ]
            mx = jnp.full((LANES,), jnp.float32(_T0))
            for base, ng in regions:
                def mit(j, m, base=base):
                    b = base + j * 64
                    m = jnp.maximum(m, keys[pl.ds(b, LANES)])
                    m = jnp.maximum(m, keys[pl.ds(b + 16, LANES)])
                    m = jnp.maximum(m, keys[pl.ds(b + 32, LANES)])
                    m = jnp.maximum(m, keys[pl.ds(b + 48, LANES)])
                    return m

                mx = lax.fori_loop(0, ng, mit, mx)
            hi0 = lax.bitcast_convert_type(jnp.max(mx), jnp.int32) + 1
            kth = find_kth(regions, jnp.int32(_T0_BITS), hi0)
            return lax.bitcast_convert_type(kth, jnp.float32)

        def fallback_kth():
            def fcond(c):
                lo, hi = c
                return hi - lo > 1

            def fbody(c):
                lo, hi = c
                mid = (
                    lax.shift_right_arithmetic(lo, 1)
                    + lax.shift_right_arithmetic(hi, 1)
                    + (lo & hi & 1)
                )
                mid_vec = jnp.full((LANES,), mid)
                one = jnp.full((LANES,), jnp.int32(1))
                zero = jnp.zeros((LANES,), jnp.int32)

                def cit(j, acc):
                    base = j * 64
                    for u in range(4):
                        bits = lax.bitcast_convert_type(
                            buf[pl.ds(base + u * LANES, LANES)], jnp.int32
                        )
                        acc = acc + jnp.where(
                            _keys_of_bits(bits) >= mid_vec, one, zero
                        )
                    return acc

                acc = lax.fori_loop(
                    0, GROUPS, cit, jnp.zeros((LANES,), jnp.int32)
                )
                big = jnp.sum(acc) >= TOPK
                return (jnp.where(big, mid, lo), jnp.where(big, hi, mid))

            kth, _ = lax.while_loop(
                fcond, fbody, (jnp.int32(_MIN_I32 + 1), jnp.int32(_HI_KEY))
            )
            return lax.bitcast_convert_type(_keys_of_bits(kth), jnp.float32)

        kthf = lax.cond(
            jnp.logical_or(c_tot < TOPK, c_max > RCAP),
            fallback_kth,
            fast_kth,
        )
        kf_vec = jnp.full((LANES,), kthf)

        copies = []
        for q in range(4):

            @pl.loop(0, SEG // 128)
            def _(i, q=q):
                base = q * SEG + i * 128
                for u in range(8):
                    sl = pl.ds(base + u * LANES, LANES)
                    v = buf[sl]
                    buf[sl] = jnp.where(v >= kf_vec, v, zero_vec)

            copies.append(
                pltpu.async_copy(
                    buf.at[pl.ds(q * SEG, SEG)],
                    o_hbm.at[row, pl.ds(q * SEG, SEG)],
                    sem_out,
                )
            )
        return copies

    cp_in0.wait()
    cps0 = process(buf0, r0, so0)
    cp_in1.wait()
    cps1 = process(buf1, r0 + 1, so1)
    for c in cps0:
        c.wait()
    for c in cps1:
        c.wait()


def kernel(x):
    mesh = plsc.VectorSubcoreMesh(core_axis_name="c", subcore_axis_name="s")
    cp = pltpu.CompilerParams()
    if "needs_layout_passes" in pltpu.CompilerParams.__dataclass_fields__:
        cp = dataclasses.replace(cp, needs_layout_passes=False)
    run = pl.kernel(
        _kernel_body,
        out_type=jax.ShapeDtypeStruct((ROWS, COLS), jnp.float32),
        mesh=mesh,
        compiler_params=cp,
        scratch_types=[
            pltpu.VMEM((COLS,), jnp.float32),
            pltpu.VMEM((COLS,), jnp.float32),
            pltpu.VMEM(((NSEG - 1) * RSTRIDE + SEG + 4 * LANES,), jnp.float32),
            pltpu.SemaphoreType.DMA,
            pltpu.SemaphoreType.DMA,
            pltpu.SemaphoreType.DMA,
            pltpu.SemaphoreType.DMA,
        ],
    )
    return run(x)
